# baseline (speedup 1.0000x reference)
import jax
import jax.numpy as jnp
from jax import lax
from jax.experimental import pallas as pl
from jax.experimental.pallas import tpu as pltpu
from jax.experimental.pallas import tpu_sc as plsc

NC = 2
NS = 16
NW = NC * NS
LANES = 16

B, S, D = 4, 2048, 768
POS_PER_W = S // NW
K = 16
NQ = POS_PER_W // K
NCH = 2 * NQ
CROWS = 2 * K
COLS = D // LANES
CHALF = COLS // 2
NBUF = 4


def _embed_body(ids_hbm, wte_hbm, wpe_hbm, out_hbm,
                idx_v, bufs, wpe_s, sem_i, sem_p, sem_g, sem_o):
  cid = lax.axis_index("c")
  sid = lax.axis_index("s")
  wid = sid * NC + cid
  pos0 = wid * POS_PER_W

  idx_cps = []
  for n in range(NCH):
    q, hb = n // 2, n % 2
    for b2 in range(2):
      idx_cps.append(pltpu.async_copy(
          ids_hbm.at[hb * 2 + b2, pl.ds(pos0 + q * K, K)],
          idx_v.at[n, pl.ds(b2 * K, K)], sem_i))

  def issue_slab(q):
    return pltpu.async_copy(
        wpe_hbm.at[pl.ds(pos0 + q * K, K)], wpe_s.at[q % 2], sem_p)

  def issue_gather(n):
    return pltpu.async_copy(
        wte_hbm.at[idx_v.at[n]], bufs.at[n % NBUF], sem_g)

  issue_slab(0)
  issue_slab(1)
  for n in range(3):
    idx_cps[2 * n].wait()
    idx_cps[2 * n + 1].wait()
    issue_gather(n)
  for cp in idx_cps[6:]:
    cp.wait()

  def chunk_body(n, _):
    gp = n % NBUF
    q = n // 2
    hb = n % 2
    sp = q % 2

    @pl.when(hb == 0)
    def _():
      pltpu.make_async_copy(
          wpe_hbm.at[pl.ds(pos0, K)], wpe_s.at[0], sem_p).wait()

    pltpu.make_async_copy(
        wte_hbm.at[idx_v.at[n]], bufs.at[gp], sem_g).wait()

    def row_body(r, _):
      for half in range(2):
        base = half * CHALF * LANES
        wrow = [wpe_s[sp, r, pl.ds(base + j * LANES, LANES)]
                for j in range(CHALF)]
        for b2 in range(2):
          row = b2 * K + r
          for j in range(CHALF):
            sl = pl.ds(base + j * LANES, LANES)
            bufs[gp, row, sl] = bufs[gp, row, sl] + wrow[j]
      return 0

    lax.fori_loop(0, K, row_body, 0)

    for b2 in range(2):
      pltpu.async_copy(
          bufs.at[gp, pl.ds(b2 * K, K)],
          out_hbm.at[pl.ds((hb * 2 + b2) * S + pos0 + q * K, K)], sem_o)

    @pl.when(jnp.logical_and(hb == 1, q + 2 < NQ))
    def _():
      pltpu.async_copy(
          wpe_hbm.at[pl.ds(pos0 + (q + 2) * K, K)], wpe_s.at[sp], sem_p)

    @pl.when(n >= 1)
    def _():
      for b2 in range(2):
        pltpu.make_async_copy(
            bufs.at[(n - 1) % NBUF, pl.ds(b2 * K, K)],
            out_hbm.at[pl.ds(pos0, K)], sem_o).wait()

    @pl.when(n + 3 < NCH)
    def _():
      pltpu.async_copy(
          wte_hbm.at[idx_v.at[n + 3]], bufs.at[(n + 3) % NBUF], sem_g)

    return 0

  lax.fori_loop(0, NCH, chunk_body, 0)

  for b2 in range(2):
    pltpu.make_async_copy(
        bufs.at[(NCH - 1) % NBUF, pl.ds(b2 * K, K)],
        out_hbm.at[pl.ds(pos0, K)], sem_o).wait()


@jax.jit
def _embed(ids, wte, wpe):
  mesh = plsc.VectorSubcoreMesh(core_axis_name="c", subcore_axis_name="s")
  f = pl.kernel(
      _embed_body,
      out_type=jax.ShapeDtypeStruct((B * S, D), jnp.float32),
      mesh=mesh,
      scratch_types=[
          pltpu.VMEM((NCH, CROWS), jnp.int32),
          pltpu.VMEM((NBUF, CROWS, D), jnp.float32),
          pltpu.VMEM((2, K, D), jnp.float32),
          pltpu.SemaphoreType.DMA,
          pltpu.SemaphoreType.DMA,
          pltpu.SemaphoreType.DMA,
          pltpu.SemaphoreType.DMA,
      ],
  )
  return f(ids, wte, wpe)


def kernel(input_ids, wte, wpe):
  out = _embed(input_ids.astype(jnp.int32), wte, wpe)
  return out.reshape(B, S, D)
